# baseline (speedup 1.0000x reference)
import functools

import jax
import jax.numpy as jnp
from jax import lax
from jax.experimental import pallas as pl
from jax.experimental.pallas import tpu as pltpu
from jax.experimental.pallas import tpu_sc as plsc

_N = 10000
_E = 320000
_D = 128
_NC = 2
_NS = 16
_NW = _NC * _NS
_C = 80
_EPW = _E // _NW
_CH = _EPW // _C
_SB = 5
_SCH = _CH // _SB
_NP = 10240
_RPT = _NP // _NS

_mesh = plsc.VectorSubcoreMesh(
    core_axis_name="c", subcore_axis_name="s", num_cores=_NC, num_subcores=_NS)


@functools.partial(
    pl.kernel,
    out_type=jax.ShapeDtypeStruct((_NC, _NP, _D), jnp.float32),
    mesh=_mesh,
    scratch_types=[
        pltpu.VMEM((_SCH, _C), jnp.int32),
        pltpu.VMEM((_SCH, _C), jnp.int32),
        pltpu.VMEM((_C, _D), jnp.float32),
        pltpu.VMEM((_C, _D), jnp.float32),
        pltpu.VMEM((_C, _D), jnp.float32),
        pltpu.VMEM((_C, _D), jnp.float32),
        pltpu.VMEM_SHARED((_NP, _D), jnp.float32),
        pltpu.SemaphoreType.DMA,
        pltpu.SemaphoreType.DMA,
        pltpu.SemaphoreType.DMA,
        pltpu.SemaphoreType.DMA,
        pltpu.SemaphoreType.DMA,
        pltpu.SemaphoreType.DMA,
        pltpu.SemaphoreType.DMA,
        pltpu.SemaphoreType.DMA,
    ],
)
def _sc_aggregate(src_hbm, dst_hbm, x_hbm, zeros_hbm, part_hbm,
                  src_v, dst_v, rows0, rows1, rows2, rows3, aggr_sh,
                  g0, g1, g2, g3, s0, s1, s2, s3):
    c = lax.axis_index("c")
    s = lax.axis_index("s")
    w = s * _NC + c
    rows = (rows0, rows1, rows2, rows3)
    gsem = (g0, g1, g2, g3)
    ssem = (s0, s1, s2, s3)

    pltpu.sync_copy(zeros_hbm.at[pl.ds(s * _RPT, _RPT)],
                    aggr_sh.at[pl.ds(s * _RPT, _RPT)])

    plsc.subcore_barrier()

    def _gather(k, slot):
        pltpu.async_copy(x_hbm.at[src_v.at[k]], rows[slot], gsem[slot])

    def _wait_scatter(k, slot):
        pltpu.make_async_copy(
            rows[slot], aggr_sh.at[dst_v.at[k]], ssem[slot]).wait()

    def _step(k, slot, do_ws, do_g):
        pltpu.make_async_copy(x_hbm.at[src_v.at[k]], rows[slot],
                              gsem[slot]).wait()
        pltpu.async_copy(rows[slot], aggr_sh.at[dst_v.at[k]], ssem[slot],
                         add=True)
        nslot = (slot + 2) % 4
        if do_ws:
            _wait_scatter(k - 2, nslot)
        if do_g:
            _gather(k + 2, nslot)

    for sb in range(_SB):
        pltpu.sync_copy(src_hbm.at[w, sb], src_v)
        pltpu.sync_copy(dst_hbm.at[w, sb], dst_v)
        _gather(0, 0)
        _gather(1, 1)
        _step(0, 0, do_ws=False, do_g=True)
        _step(1, 1, do_ws=False, do_g=True)

        def _quad(t, cc):
            k = 2 + 4 * t
            _step(k, 2, do_ws=True, do_g=True)
            _step(k + 1, 3, do_ws=True, do_g=True)
            _step(k + 2, 0, do_ws=True, do_g=True)
            _step(k + 3, 1, do_ws=True, do_g=True)
            return cc

        lax.fori_loop(0, 5, _quad, 0)

        _step(_SCH - 3, 2, do_ws=True, do_g=True)
        _step(_SCH - 2, 3, do_ws=True, do_g=False)
        _step(_SCH - 1, 0, do_ws=True, do_g=False)
        _wait_scatter(_SCH - 2, 3)
        _wait_scatter(_SCH - 1, 0)

    plsc.subcore_barrier()

    pltpu.sync_copy(aggr_sh.at[pl.ds(s * _RPT, _RPT)],
                    part_hbm.at[c, pl.ds(s * _RPT, _RPT)])


def _tc_body(x_ref, p_ref, w1_ref, w2_ref, b1_ref, b2_ref, o_ref):
    cdims = (((1,), (1,)), ((), ()))
    y = lax.dot_general(x_ref[...], w1_ref[...], cdims,
                        preferred_element_type=jnp.float32)
    aggr = p_ref[0] + p_ref[1]
    y = y + lax.dot_general(aggr, w2_ref[...], cdims,
                            preferred_element_type=jnp.float32)
    o_ref[...] = y + b1_ref[...] + b2_ref[...]


_R = 2000


def _tc_combine(x, partials, W1, W2, b1_2d, b2_2d):
    grid = (_N // _R,)
    return pl.pallas_call(
        _tc_body,
        out_shape=jax.ShapeDtypeStruct((_N, _D), jnp.float32),
        grid=grid,
        in_specs=[
            pl.BlockSpec((_R, _D), lambda i: (i, 0)),
            pl.BlockSpec((_NC, _R, _D), lambda i: (0, i, 0)),
            pl.BlockSpec((_D, _D), lambda i: (0, 0)),
            pl.BlockSpec((_D, _D), lambda i: (0, 0)),
            pl.BlockSpec((1, _D), lambda i: (0, 0)),
            pl.BlockSpec((1, _D), lambda i: (0, 0)),
        ],
        out_specs=pl.BlockSpec((_R, _D), lambda i: (i, 0)),
    )(x, partials, W1, W2, b1_2d, b2_2d)


def kernel(edge_index, shape_features, W1, b1, W2, b2):
    src3d = edge_index[0].reshape(_NW, _SB, _SCH, _C)
    dst3d = edge_index[1].reshape(_NW, _SB, _SCH, _C)
    zeros = jnp.zeros((_NP, _D), jnp.float32)
    partials = _sc_aggregate(src3d, dst3d, shape_features, zeros)
    return _tc_combine(shape_features, partials, W1, W2,
                       b1.reshape(1, _D), b2.reshape(1, _D))


# speedup vs baseline: 1.2592x; 1.2592x over previous
import functools

import jax
import jax.numpy as jnp
from jax import lax
from jax.experimental import pallas as pl
from jax.experimental.pallas import tpu as pltpu
from jax.experimental.pallas import tpu_sc as plsc

_N = 10000
_E = 320000
_D = 128
_NC = 2
_NS = 16
_NW = _NC * _NS
_C = 80
_EPW = _E // _NW
_CH = _EPW // _C
_SB = 5
_SCH = _CH // _SB
_NP = 10240
_RPT = _NP // _NS

_mesh = plsc.VectorSubcoreMesh(
    core_axis_name="c", subcore_axis_name="s", num_cores=_NC, num_subcores=_NS)


@functools.partial(
    pl.kernel,
    out_type=jax.ShapeDtypeStruct((_NC, _NP, _D), jnp.float32),
    mesh=_mesh,
    scratch_types=[
        pltpu.VMEM((_SCH, _C), jnp.int32),
        pltpu.VMEM((_SCH, _C), jnp.int32),
        pltpu.VMEM((_SCH, _C), jnp.int32),
        pltpu.VMEM((_SCH, _C), jnp.int32),
        pltpu.VMEM((_C, _D), jnp.float32),
        pltpu.VMEM((_C, _D), jnp.float32),
        pltpu.VMEM((_C, _D), jnp.float32),
        pltpu.VMEM_SHARED((_NP, _D), jnp.float32),
        pltpu.SemaphoreType.DMA,
        pltpu.SemaphoreType.DMA,
        pltpu.SemaphoreType.DMA,
        pltpu.SemaphoreType.DMA,
        pltpu.SemaphoreType.DMA,
        pltpu.SemaphoreType.DMA,
        pltpu.SemaphoreType.DMA,
        pltpu.SemaphoreType.DMA,
    ],
)
def _sc_aggregate(edges_hbm, x_hbm, part_hbm,
                  src_a, dst_a, src_b, dst_b, rows0, rows1, rows2, aggr_sh,
                  g0, g1, g2, s0, s1, s2, ia, ib):
    c = lax.axis_index("c")
    s = lax.axis_index("s")
    w = s * _NC + c
    rows = (rows0, rows1, rows2)
    gsem = (g0, g1, g2)
    ssem = (s0, s1, s2)
    idx_sets = ((src_a, dst_a, ia), (src_b, dst_b, ib))

    def _zrow(r, cc):
        def _zq(q, c2):
            rows0[r, pl.ds(q * 16, 16)] = jnp.zeros((16,), jnp.float32)
            return c2
        return lax.fori_loop(0, _D // 16, _zq, cc)

    lax.fori_loop(0, _C, _zrow, 0)
    for j in range(_RPT // _C):
        pltpu.sync_copy(rows0, aggr_sh.at[pl.ds(s * _RPT + j * _C, _C)])

    plsc.subcore_barrier()

    def _idx_fetch(sb, st):
        src_v, dst_v, isem = idx_sets[st]
        pltpu.async_copy(edges_hbm.at[0, w, sb], src_v, isem)
        pltpu.async_copy(edges_hbm.at[1, w, sb], dst_v, isem)

    def _idx_wait(sb, st):
        src_v, dst_v, isem = idx_sets[st]
        pltpu.make_async_copy(edges_hbm.at[0, w, sb], src_v, isem).wait()
        pltpu.make_async_copy(edges_hbm.at[1, w, sb], dst_v, isem).wait()

    _idx_fetch(0, 0)

    for sb in range(_SB):
        src_v, dst_v, _ = idx_sets[sb % 2]

        def _gather(k, slot):
            pltpu.async_copy(x_hbm.at[src_v.at[k]], rows[slot], gsem[slot])

        def _wait_scatter(k, slot):
            pltpu.make_async_copy(
                rows[slot], aggr_sh.at[dst_v.at[k]], ssem[slot]).wait()

        def _step(k, slot, do_ws, do_g):
            pltpu.make_async_copy(x_hbm.at[src_v.at[k]], rows[slot],
                                  gsem[slot]).wait()
            pltpu.async_copy(rows[slot], aggr_sh.at[dst_v.at[k]],
                             ssem[slot], add=True)
            if do_ws:
                _wait_scatter(k - 1, (slot + 2) % 3)
            if do_g:
                _gather(k + 2, (slot + 2) % 3)

        _idx_wait(sb, sb % 2)
        if sb + 1 < _SB:
            _idx_fetch(sb + 1, (sb + 1) % 2)
        _gather(0, 0)
        _gather(1, 1)
        _step(0, 0, do_ws=False, do_g=True)
        _step(1, 1, do_ws=True, do_g=True)

        def _trip(t, cc):
            k = 2 + 3 * t
            _step(k, 2, do_ws=True, do_g=True)
            _step(k + 1, 0, do_ws=True, do_g=True)
            _step(k + 2, 1, do_ws=True, do_g=True)
            return cc

        lax.fori_loop(0, (_SCH - 4) // 3, _trip, 0)

        _step(_SCH - 2, 2, do_ws=True, do_g=False)
        _step(_SCH - 1, 0, do_ws=True, do_g=False)
        _wait_scatter(_SCH - 1, 0)

    plsc.subcore_barrier()

    pltpu.sync_copy(aggr_sh.at[pl.ds(s * _RPT, _RPT)],
                    part_hbm.at[c, pl.ds(s * _RPT, _RPT)])


def _tc_body(x_ref, p_ref, w1_ref, w2_ref, b1_ref, b2_ref, o_ref):
    cdims = (((1,), (1,)), ((), ()))
    y = lax.dot_general(x_ref[...], w1_ref[...], cdims,
                        preferred_element_type=jnp.float32)
    aggr = p_ref[0] + p_ref[1]
    y = y + lax.dot_general(aggr, w2_ref[...], cdims,
                            preferred_element_type=jnp.float32)
    o_ref[...] = y + b1_ref[...] + b2_ref[...]


_R = 2000


def _tc_combine(x, partials, W1, W2, b1_2d, b2_2d):
    grid = (_N // _R,)
    return pl.pallas_call(
        _tc_body,
        out_shape=jax.ShapeDtypeStruct((_N, _D), jnp.float32),
        grid=grid,
        in_specs=[
            pl.BlockSpec((_R, _D), lambda i: (i, 0)),
            pl.BlockSpec((_NC, _R, _D), lambda i: (0, i, 0)),
            pl.BlockSpec((_D, _D), lambda i: (0, 0)),
            pl.BlockSpec((_D, _D), lambda i: (0, 0)),
            pl.BlockSpec((1, _D), lambda i: (0, 0)),
            pl.BlockSpec((1, _D), lambda i: (0, 0)),
        ],
        out_specs=pl.BlockSpec((_R, _D), lambda i: (i, 0)),
    )(x, partials, W1, W2, b1_2d, b2_2d)


def kernel(edge_index, shape_features, W1, b1, W2, b2):
    edges5d = edge_index.reshape(2, _NW, _SB, _SCH, _C)
    partials = _sc_aggregate(edges5d, shape_features)
    return _tc_combine(shape_features, partials, W1, W2,
                       b1.reshape(1, _D), b2.reshape(1, _D))


# speedup vs baseline: 1.3013x; 1.0335x over previous
import functools

import jax
import jax.numpy as jnp
from jax import lax
from jax.experimental import pallas as pl
from jax.experimental.pallas import tpu as pltpu
from jax.experimental.pallas import tpu_sc as plsc

_N = 10000
_E = 320000
_D = 128
_NC = 2
_NS = 16
_NW = _NC * _NS
_C = 80
_EPW = _E // _NW
_CH = _EPW // _C
_SB = 5
_SCH = _CH // _SB
_NP = 10240
_RPT = _NP // _NS

_mesh = plsc.VectorSubcoreMesh(
    core_axis_name="c", subcore_axis_name="s", num_cores=_NC, num_subcores=_NS)


@functools.partial(
    pl.kernel,
    out_type=jax.ShapeDtypeStruct((_NC, _NP, _D), jnp.float32),
    mesh=_mesh,
    scratch_types=[
        pltpu.VMEM((_SCH, _C), jnp.int32),
        pltpu.VMEM((_SCH, _C), jnp.int32),
        pltpu.VMEM((_SCH, _C), jnp.int32),
        pltpu.VMEM((_SCH, _C), jnp.int32),
        pltpu.VMEM((_C, _D), jnp.float32),
        pltpu.VMEM((_C, _D), jnp.float32),
        pltpu.VMEM((_C, _D), jnp.float32),
        pltpu.VMEM_SHARED((_NP, _D), jnp.float32),
        pltpu.SemaphoreType.DMA,
        pltpu.SemaphoreType.DMA,
        pltpu.SemaphoreType.DMA,
        pltpu.SemaphoreType.DMA,
        pltpu.SemaphoreType.DMA,
        pltpu.SemaphoreType.DMA,
        pltpu.SemaphoreType.DMA,
        pltpu.SemaphoreType.DMA,
    ],
)
def _sc_aggregate(edges_hbm, x_hbm, part_hbm,
                  src_a, dst_a, src_b, dst_b, rows0, rows1, rows2, aggr_sh,
                  g0, g1, g2, s0, s1, s2, ia, ib):
    c = lax.axis_index("c")
    s = lax.axis_index("s")
    w = s * _NC + c
    rows = (rows0, rows1, rows2)
    gsem = (g0, g1, g2)
    ssem = (s0, s1, s2)
    idx_sets = ((src_a, dst_a, ia), (src_b, dst_b, ib))

    def _zrow(r, cc):
        def _zq(q, c2):
            rows0[r, pl.ds(q * 16, 16)] = jnp.zeros((16,), jnp.float32)
            return c2
        return lax.fori_loop(0, _D // 16, _zq, cc)

    lax.fori_loop(0, _C, _zrow, 0)
    for j in range(_RPT // _C):
        pltpu.sync_copy(rows0, aggr_sh.at[pl.ds(s * _RPT + j * _C, _C)])

    plsc.subcore_barrier()

    def _idx_fetch(sb, st):
        src_v, dst_v, isem = idx_sets[st]
        pltpu.async_copy(edges_hbm.at[0, w, sb], src_v, isem)
        pltpu.async_copy(edges_hbm.at[1, w, sb], dst_v, isem)

    def _idx_wait(sb, st):
        src_v, dst_v, isem = idx_sets[st]
        pltpu.make_async_copy(edges_hbm.at[0, w, sb], src_v, isem).wait()
        pltpu.make_async_copy(edges_hbm.at[1, w, sb], dst_v, isem).wait()

    def _src(sb):
        return idx_sets[sb % 2][0]

    def _dst(sb):
        return idx_sets[sb % 2][1]

    def _gather(sb, k, slot):
        pltpu.async_copy(x_hbm.at[_src(sb).at[k]], rows[slot], gsem[slot])

    def _wait_scatter(sb, k, slot):
        pltpu.make_async_copy(
            rows[slot], aggr_sh.at[_dst(sb).at[k]], ssem[slot]).wait()

    def _step(sb, k, do_ws=True, do_g=True):
        g = _SCH * sb + k
        slot = g % 3
        pltpu.make_async_copy(x_hbm.at[_src(sb).at[k]], rows[slot],
                              gsem[slot]).wait()
        pltpu.async_copy(rows[slot], aggr_sh.at[_dst(sb).at[k]],
                         ssem[slot], add=True)
        if do_ws:
            pg = g - 1
            _wait_scatter(pg // _SCH, pg % _SCH, pg % 3)
        if do_g:
            ng = g + 2
            _gather(ng // _SCH, ng % _SCH, ng % 3)

    _idx_fetch(0, 0)
    _idx_wait(0, 0)
    _idx_fetch(1, 1)
    _gather(0, 0, 0)
    _gather(0, 1, 1)
    _step(0, 0, do_ws=False)
    _step(0, 1)

    for sb in range(_SB):
        def _trip(t, cc, sb=sb):
            k = 2 + 3 * t
            base = (_SCH * sb + 2) % 3
            for j in range(3):
                g = _SCH * sb + k + j
                slot = (base + j) % 3
                pltpu.make_async_copy(x_hbm.at[_src(sb).at[k + j]],
                                      rows[slot], gsem[slot]).wait()
                pltpu.async_copy(rows[slot], aggr_sh.at[_dst(sb).at[k + j]],
                                 ssem[slot], add=True)
                pltpu.make_async_copy(rows[(slot + 2) % 3],
                                      aggr_sh.at[_dst(sb).at[k + j - 1]],
                                      ssem[(slot + 2) % 3]).wait()
                pltpu.async_copy(x_hbm.at[_src(sb).at[k + j + 2]],
                                 rows[(slot + 2) % 3], gsem[(slot + 2) % 3])
            return cc

        lax.fori_loop(0, (_SCH - 4) // 3, _trip, 0)

        if sb + 1 < _SB:
            _idx_wait(sb + 1, (sb + 1) % 2)
            _step(sb, _SCH - 2)
            _step(sb, _SCH - 1)
            _step(sb + 1, 0)
            _step(sb + 1, 1)
            if sb + 2 < _SB:
                _idx_fetch(sb + 2, sb % 2)
        else:
            _step(sb, _SCH - 2, do_g=False)
            _step(sb, _SCH - 1, do_g=False)
            _wait_scatter(sb, _SCH - 1, (_SCH * sb + _SCH - 1) % 3)

    plsc.subcore_barrier()

    pltpu.sync_copy(aggr_sh.at[pl.ds(s * _RPT, _RPT)],
                    part_hbm.at[c, pl.ds(s * _RPT, _RPT)])


def _tc_body(x_ref, p_ref, w1_ref, w2_ref, b1_ref, b2_ref, o_ref):
    cdims = (((1,), (1,)), ((), ()))
    y = lax.dot_general(x_ref[...], w1_ref[...], cdims,
                        preferred_element_type=jnp.float32)
    aggr = p_ref[0] + p_ref[1]
    y = y + lax.dot_general(aggr, w2_ref[...], cdims,
                            preferred_element_type=jnp.float32)
    o_ref[...] = y + b1_ref[...] + b2_ref[...]


_R = 2000


def _tc_combine(x, partials, W1, W2, b1_2d, b2_2d):
    grid = (_N // _R,)
    return pl.pallas_call(
        _tc_body,
        out_shape=jax.ShapeDtypeStruct((_N, _D), jnp.float32),
        grid=grid,
        in_specs=[
            pl.BlockSpec((_R, _D), lambda i: (i, 0)),
            pl.BlockSpec((_NC, _R, _D), lambda i: (0, i, 0)),
            pl.BlockSpec((_D, _D), lambda i: (0, 0)),
            pl.BlockSpec((_D, _D), lambda i: (0, 0)),
            pl.BlockSpec((1, _D), lambda i: (0, 0)),
            pl.BlockSpec((1, _D), lambda i: (0, 0)),
        ],
        out_specs=pl.BlockSpec((_R, _D), lambda i: (i, 0)),
    )(x, partials, W1, W2, b1_2d, b2_2d)


def kernel(edge_index, shape_features, W1, b1, W2, b2):
    edges5d = edge_index.reshape(2, _NW, _SB, _SCH, _C)
    partials = _sc_aggregate(edges5d, shape_features)
    return _tc_combine(shape_features, partials, W1, W2,
                       b1.reshape(1, _D), b2.reshape(1, _D))


# speedup vs baseline: 1.3043x; 1.0022x over previous
import functools

import jax
import jax.numpy as jnp
from jax import lax
from jax.experimental import pallas as pl
from jax.experimental.pallas import tpu as pltpu
from jax.experimental.pallas import tpu_sc as plsc

_N = 10000
_E = 320000
_D = 128
_NC = 2
_NS = 16
_NW = _NC * _NS
_C = 80
_EPW = _E // _NW
_CH = _EPW // _C
_SB = 5
_SCH = _CH // _SB
_NP = 10240
_RPT = _NP // _NS

_mesh = plsc.VectorSubcoreMesh(
    core_axis_name="c", subcore_axis_name="s", num_cores=_NC, num_subcores=_NS)


@functools.partial(
    pl.kernel,
    out_type=jax.ShapeDtypeStruct((_NC, _NP, _D), jnp.float32),
    mesh=_mesh,
    scratch_types=[
        pltpu.VMEM((_SCH, _C), jnp.int32),
        pltpu.VMEM((_SCH, _C), jnp.int32),
        pltpu.VMEM((_SCH, _C), jnp.int32),
        pltpu.VMEM((_SCH, _C), jnp.int32),
        pltpu.VMEM((_C, _D), jnp.float32),
        pltpu.VMEM((_C, _D), jnp.float32),
        pltpu.VMEM((_C, _D), jnp.float32),
        pltpu.VMEM_SHARED((_NP, _D), jnp.float32),
        pltpu.SemaphoreType.DMA,
        pltpu.SemaphoreType.DMA,
        pltpu.SemaphoreType.DMA,
        pltpu.SemaphoreType.DMA,
        pltpu.SemaphoreType.DMA,
        pltpu.SemaphoreType.DMA,
        pltpu.SemaphoreType.DMA,
        pltpu.SemaphoreType.DMA,
    ],
)
def _sc_aggregate(edges_hbm, x_hbm, part_hbm,
                  src_a, dst_a, src_b, dst_b, rows0, rows1, rows2, aggr_sh,
                  g0, g1, g2, s0, s1, s2, ia, ib):
    c = lax.axis_index("c")
    s = lax.axis_index("s")
    w = s * _NC + c
    rows = (rows0, rows1, rows2)
    gsem = (g0, g1, g2)
    ssem = (s0, s1, s2)
    idx_sets = ((src_a, dst_a, ia), (src_b, dst_b, ib))

    def _zrow(r, cc):
        def _zq(q, c2):
            rows0[r, pl.ds(q * 16, 16)] = jnp.zeros((16,), jnp.float32)
            return c2
        return lax.fori_loop(0, _D // 16, _zq, cc)

    lax.fori_loop(0, _C, _zrow, 0)
    for j in range(_RPT // _C):
        pltpu.sync_copy(rows0, aggr_sh.at[pl.ds(s * _RPT + j * _C, _C)])

    plsc.subcore_barrier()

    def _idx_fetch(sb, st):
        src_v, dst_v, isem = idx_sets[st]
        pltpu.async_copy(edges_hbm.at[0, w, sb], src_v, isem)
        pltpu.async_copy(edges_hbm.at[1, w, sb], dst_v, isem)

    def _idx_wait(sb, st):
        src_v, dst_v, isem = idx_sets[st]
        pltpu.make_async_copy(edges_hbm.at[0, w, sb], src_v, isem).wait()
        pltpu.make_async_copy(edges_hbm.at[1, w, sb], dst_v, isem).wait()

    def _src(sb):
        return idx_sets[sb % 2][0]

    def _dst(sb):
        return idx_sets[sb % 2][1]

    def _gather(sb, k, slot):
        pltpu.async_copy(x_hbm.at[_src(sb).at[k]], rows[slot], gsem[slot])

    def _wait_scatter(sb, k, slot):
        pltpu.make_async_copy(
            rows[slot], aggr_sh.at[_dst(sb).at[k]], ssem[slot]).wait()

    def _step(sb, k, do_ws=True, do_g=True):
        g = _SCH * sb + k
        slot = g % 3
        pltpu.make_async_copy(x_hbm.at[_src(sb).at[k]], rows[slot],
                              gsem[slot]).wait()
        pltpu.async_copy(rows[slot], aggr_sh.at[_dst(sb).at[k]],
                         ssem[slot], add=True)
        if do_ws:
            pg = g - 1
            _wait_scatter(pg // _SCH, pg % _SCH, pg % 3)
        if do_g:
            ng = g + 2
            _gather(ng // _SCH, ng % _SCH, ng % 3)

    _idx_fetch(0, 0)
    _idx_wait(0, 0)
    _idx_fetch(1, 1)
    _gather(0, 0, 0)
    _gather(0, 1, 1)
    _step(0, 0, do_ws=False)
    _step(0, 1)

    for sb in range(_SB):
        def _trip(t, cc, sb=sb):
            k = 2 + 3 * t
            base = (_SCH * sb + 2) % 3
            for j in range(3):
                g = _SCH * sb + k + j
                slot = (base + j) % 3
                pltpu.make_async_copy(x_hbm.at[_src(sb).at[k + j]],
                                      rows[slot], gsem[slot]).wait()
                pltpu.async_copy(rows[slot], aggr_sh.at[_dst(sb).at[k + j]],
                                 ssem[slot], add=True)
                pltpu.make_async_copy(rows[(slot + 2) % 3],
                                      aggr_sh.at[_dst(sb).at[k + j - 1]],
                                      ssem[(slot + 2) % 3]).wait()
                pltpu.async_copy(x_hbm.at[_src(sb).at[k + j + 2]],
                                 rows[(slot + 2) % 3], gsem[(slot + 2) % 3])
            return cc

        lax.fori_loop(0, (_SCH - 4) // 3, _trip, 0)

        if sb + 1 < _SB:
            _idx_wait(sb + 1, (sb + 1) % 2)
            _step(sb, _SCH - 2)
            _step(sb, _SCH - 1)
            _step(sb + 1, 0)
            _step(sb + 1, 1)
            if sb + 2 < _SB:
                _idx_fetch(sb + 2, sb % 2)
        else:
            _step(sb, _SCH - 2, do_g=False)
            _step(sb, _SCH - 1, do_g=False)
            _wait_scatter(sb, _SCH - 1, (_SCH * sb + _SCH - 1) % 3)

    plsc.subcore_barrier()

    pltpu.sync_copy(aggr_sh.at[pl.ds(s * _RPT, _RPT)],
                    part_hbm.at[c, pl.ds(s * _RPT, _RPT)])


_R = 2000


def _tc_xw1_body(x_ref, w1_ref, b1_ref, b2_ref, o_ref):
    cdims = (((1,), (1,)), ((), ()))
    y = lax.dot_general(x_ref[...], w1_ref[...], cdims,
                        preferred_element_type=jnp.float32)
    o_ref[...] = y + b1_ref[...] + b2_ref[...]


def _tc_xw1(x, W1, b1_2d, b2_2d):
    return pl.pallas_call(
        _tc_xw1_body,
        out_shape=jax.ShapeDtypeStruct((_N, _D), jnp.float32),
        grid=(_N // _R,),
        in_specs=[
            pl.BlockSpec((_R, _D), lambda i: (i, 0)),
            pl.BlockSpec((_D, _D), lambda i: (0, 0)),
            pl.BlockSpec((1, _D), lambda i: (0, 0)),
            pl.BlockSpec((1, _D), lambda i: (0, 0)),
        ],
        out_specs=pl.BlockSpec((_R, _D), lambda i: (i, 0)),
    )(x, W1, b1_2d, b2_2d)


def _tc_combine_body(y1_ref, p_ref, w2_ref, o_ref):
    cdims = (((1,), (1,)), ((), ()))
    aggr = p_ref[0] + p_ref[1]
    o_ref[...] = y1_ref[...] + lax.dot_general(
        aggr, w2_ref[...], cdims, preferred_element_type=jnp.float32)


def _tc_combine(y1, partials, W2):
    return pl.pallas_call(
        _tc_combine_body,
        out_shape=jax.ShapeDtypeStruct((_N, _D), jnp.float32),
        grid=(_N // _R,),
        in_specs=[
            pl.BlockSpec((_R, _D), lambda i: (i, 0)),
            pl.BlockSpec((_NC, _R, _D), lambda i: (0, i, 0)),
            pl.BlockSpec((_D, _D), lambda i: (0, 0)),
        ],
        out_specs=pl.BlockSpec((_R, _D), lambda i: (i, 0)),
    )(y1, partials, W2)


def kernel(edge_index, shape_features, W1, b1, W2, b2):
    edges5d = edge_index.reshape(2, _NW, _SB, _SCH, _C)
    partials = _sc_aggregate(edges5d, shape_features)
    y1 = _tc_xw1(shape_features, W1, b1.reshape(1, _D), b2.reshape(1, _D))
    return _tc_combine(y1, partials, W2)


# speedup vs baseline: 1.3065x; 1.0017x over previous
import functools

import jax
import jax.numpy as jnp
from jax import lax
from jax.experimental import pallas as pl
from jax.experimental.pallas import tpu as pltpu
from jax.experimental.pallas import tpu_sc as plsc

_N = 10000
_E = 320000
_D = 128
_NC = 2
_NS = 16
_NW = _NC * _NS
_C = 80
_EPW = _E // _NW
_CH = _EPW // _C
_SB = 5
_SCH = _CH // _SB
_NP = 10240
_RPT = _NP // _NS

_mesh = plsc.VectorSubcoreMesh(
    core_axis_name="c", subcore_axis_name="s", num_cores=_NC, num_subcores=_NS)


@functools.partial(
    pl.kernel,
    out_type=jax.ShapeDtypeStruct((_NC, _NP, _D), jnp.float32),
    mesh=_mesh,
    scratch_types=[
        pltpu.VMEM((_SCH, _C), jnp.int32),
        pltpu.VMEM((_SCH, _C), jnp.int32),
        pltpu.VMEM((_SCH, _C), jnp.int32),
        pltpu.VMEM((_SCH, _C), jnp.int32),
        pltpu.VMEM((_C, _D), jnp.float32),
        pltpu.VMEM((_C, _D), jnp.float32),
        pltpu.VMEM((_C, _D), jnp.float32),
        pltpu.VMEM_SHARED((_NP, _D), jnp.float32),
        pltpu.SemaphoreType.DMA,
        pltpu.SemaphoreType.DMA,
        pltpu.SemaphoreType.DMA,
        pltpu.SemaphoreType.DMA,
        pltpu.SemaphoreType.DMA,
        pltpu.SemaphoreType.DMA,
        pltpu.SemaphoreType.DMA,
        pltpu.SemaphoreType.DMA,
    ],
)
def _sc_aggregate(edges_hbm, x_hbm, part_hbm,
                  src_a, dst_a, src_b, dst_b,
                  rows0, rows1, rows2, aggr_sh,
                  g0, g1, g2, s0, s1, s2, ia, ib):
    c = lax.axis_index("c")
    s = lax.axis_index("s")
    w = s * _NC + c
    rows = (rows0, rows1, rows2)
    gsem = (g0, g1, g2)
    ssem = (s0, s1, s2)
    idx_sets = ((src_a, dst_a, ia), (src_b, dst_b, ib))

    def _zrow(r, cc):
        def _zq(q, c2):
            rows0[r, pl.ds(q * 16, 16)] = jnp.zeros((16,), jnp.float32)
            return c2
        return lax.fori_loop(0, _D // 16, _zq, cc)

    lax.fori_loop(0, _C, _zrow, 0)
    for j in range(_RPT // _C):
        pltpu.sync_copy(rows0, aggr_sh.at[pl.ds(s * _RPT + j * _C, _C)])
    if _RPT % _C:
        pltpu.sync_copy(
            rows0.at[pl.ds(0, _RPT % _C)],
            aggr_sh.at[pl.ds(s * _RPT + (_RPT // _C) * _C, _RPT % _C)])

    plsc.subcore_barrier()

    def _idx_fetch(sb, st):
        src_v, dst_v, isem = idx_sets[st]
        pltpu.async_copy(edges_hbm.at[0, w, sb], src_v, isem)
        pltpu.async_copy(edges_hbm.at[1, w, sb], dst_v, isem)

    def _idx_wait(sb, st):
        src_v, dst_v, isem = idx_sets[st]
        pltpu.make_async_copy(edges_hbm.at[0, w, sb], src_v, isem).wait()
        pltpu.make_async_copy(edges_hbm.at[1, w, sb], dst_v, isem).wait()

    def _src(sb):
        return idx_sets[sb % 2][0]

    def _dst(sb):
        return idx_sets[sb % 2][1]

    def _gather(sb, k, slot):
        pltpu.async_copy(x_hbm.at[_src(sb).at[k]], rows[slot], gsem[slot])

    def _wait_scatter(sb, k, slot):
        pltpu.make_async_copy(
            rows[slot], aggr_sh.at[_dst(sb).at[k]], ssem[slot]).wait()

    def _step(sb, k, do_ws=True, do_g=True):
        g = _SCH * sb + k
        slot = g % 3
        pltpu.make_async_copy(x_hbm.at[_src(sb).at[k]], rows[slot],
                              gsem[slot]).wait()
        pltpu.async_copy(rows[slot], aggr_sh.at[_dst(sb).at[k]],
                         ssem[slot], add=True)
        if do_ws:
            pg = g - 1
            _wait_scatter(pg // _SCH, pg % _SCH, pg % 3)
        if do_g:
            ng = g + 2
            _gather(ng // _SCH, ng % _SCH, ng % 3)

    _idx_fetch(0, 0)
    _idx_wait(0, 0)
    _idx_fetch(1, 1)
    _gather(0, 0, 0)
    _gather(0, 1, 1)
    _step(0, 0, do_ws=False)
    _step(0, 1)

    for sb in range(_SB):
        def _trip(t, cc, sb=sb):
            k = 2 + 3 * t
            base = (_SCH * sb + 2) % 3
            for j in range(3):
                slot = (base + j) % 3
                pltpu.make_async_copy(x_hbm.at[_src(sb).at[k + j]],
                                      rows[slot], gsem[slot]).wait()
                pltpu.async_copy(rows[slot], aggr_sh.at[_dst(sb).at[k + j]],
                                 ssem[slot], add=True)
                pltpu.make_async_copy(rows[(slot + 2) % 3],
                                      aggr_sh.at[_dst(sb).at[k + j - 1]],
                                      ssem[(slot + 2) % 3]).wait()
                pltpu.async_copy(x_hbm.at[_src(sb).at[k + j + 2]],
                                 rows[(slot + 2) % 3], gsem[(slot + 2) % 3])
            return cc

        lax.fori_loop(0, (_SCH - 4) // 3, _trip, 0)

        if sb + 1 < _SB:
            _idx_wait(sb + 1, (sb + 1) % 2)
            _step(sb, _SCH - 2)
            _step(sb, _SCH - 1)
            _step(sb + 1, 0)
            _step(sb + 1, 1)
            if sb + 2 < _SB:
                _idx_fetch(sb + 2, sb % 2)
        else:
            _step(sb, _SCH - 2, do_g=False)
            _step(sb, _SCH - 1, do_g=False)
            _wait_scatter(sb, _SCH - 1, (_SCH * sb + _SCH - 1) % 3)

    plsc.subcore_barrier()

    pltpu.sync_copy(aggr_sh.at[pl.ds(s * _RPT, _RPT)],
                    part_hbm.at[c, pl.ds(s * _RPT, _RPT)])


_R = 2000


def _tc_xw1_body(x_ref, w1_ref, b1_ref, b2_ref, o_ref):
    cdims = (((1,), (1,)), ((), ()))
    y = lax.dot_general(x_ref[...], w1_ref[...], cdims,
                        preferred_element_type=jnp.float32)
    o_ref[...] = y + b1_ref[...] + b2_ref[...]


def _tc_xw1(x, W1, b1_2d, b2_2d):
    return pl.pallas_call(
        _tc_xw1_body,
        out_shape=jax.ShapeDtypeStruct((_N, _D), jnp.float32),
        grid=(_N // _R,),
        in_specs=[
            pl.BlockSpec((_R, _D), lambda i: (i, 0)),
            pl.BlockSpec((_D, _D), lambda i: (0, 0)),
            pl.BlockSpec((1, _D), lambda i: (0, 0)),
            pl.BlockSpec((1, _D), lambda i: (0, 0)),
        ],
        out_specs=pl.BlockSpec((_R, _D), lambda i: (i, 0)),
    )(x, W1, b1_2d, b2_2d)


def _tc_combine_body(y1_ref, p_ref, w2_ref, o_ref):
    cdims = (((1,), (1,)), ((), ()))
    aggr = p_ref[0] + p_ref[1]
    o_ref[...] = y1_ref[...] + lax.dot_general(
        aggr, w2_ref[...], cdims, preferred_element_type=jnp.float32)


def _tc_combine(y1, partials, W2):
    return pl.pallas_call(
        _tc_combine_body,
        out_shape=jax.ShapeDtypeStruct((_N, _D), jnp.float32),
        grid=(_N // _R,),
        in_specs=[
            pl.BlockSpec((_R, _D), lambda i: (i, 0)),
            pl.BlockSpec((_NC, _R, _D), lambda i: (0, i, 0)),
            pl.BlockSpec((_D, _D), lambda i: (0, 0)),
        ],
        out_specs=pl.BlockSpec((_R, _D), lambda i: (i, 0)),
    )(y1, partials, W2)


def kernel(edge_index, shape_features, W1, b1, W2, b2):
    edges5d = edge_index.reshape(2, _NW, _SB, _SCH, _C)
    partials = _sc_aggregate(edges5d, shape_features)
    y1 = _tc_xw1(shape_features, W1, b1.reshape(1, _D), b2.reshape(1, _D))
    return _tc_combine(y1, partials, W2)
